# Initial kernel scaffold; baseline (speedup 1.0000x reference)
import jax
import jax.numpy as jnp
from jax.experimental import pallas as pl


def kernel(input, router_weight, w1, w2):
    raise NotImplementedError("write your pallas kernel here")


# speedup vs baseline: 1.1365x; 1.1365x over previous
import functools

import jax
import jax.numpy as jnp
from jax import lax
from jax.experimental import pallas as pl
from jax.experimental.pallas import tpu as pltpu
from jax.experimental.pallas import tpu_sc as plsc

E = 8
K = 2
H = 1024
F = 4096
T = 2048
CAP = 1024
TRASH = E * CAP
TBL = E * CAP + 8

NW = 32
TPW = T // NW
SUB = 32

BM = 256
FB = 2
F_BLK = F // FB
M_BLKS = CAP // BM


def _router_body(x_ref, rw_ref, meta_ref, gb0_ref, gb1_ref):
    x = x_ref[...]
    rw = rw_ref[...]
    logits = jnp.dot(x, rw, preferred_element_type=jnp.float32)
    mx = jnp.max(logits, axis=1, keepdims=True)
    ex = jnp.exp(logits - mx)
    p = ex / jnp.sum(ex, axis=1, keepdims=True)

    lane = lax.broadcasted_iota(jnp.int32, (T, E), 1)
    m1 = jnp.max(p, axis=1, keepdims=True)
    i1 = jnp.min(jnp.where(p == m1, lane, E), axis=1, keepdims=True)
    p2 = jnp.where(lane == i1, -jnp.inf, p)
    m2 = jnp.max(p2, axis=1, keepdims=True)
    i2 = jnp.min(jnp.where(p2 == m2, lane, E), axis=1, keepdims=True)
    den = m1 + m2
    g1 = m1 / den
    g2 = m2 / den

    cnt = (lane == i1).astype(jnp.float32) + (lane == i2).astype(jnp.float32)
    ri = lax.broadcasted_iota(jnp.int32, (T, T), 0)
    ci = lax.broadcasted_iota(jnp.int32, (T, T), 1)
    tri = (ci < ri).astype(jnp.float32)
    csum = jnp.dot(tri, cnt, preferred_element_type=jnp.float32)
    pos1 = jnp.sum(jnp.where(lane == i1, csum, 0.0), axis=1,
                   keepdims=True).astype(jnp.int32)
    pos2 = jnp.sum(jnp.where(lane == i2, csum, 0.0), axis=1,
                   keepdims=True).astype(jnp.int32)

    keep1 = pos1 < CAP
    keep2 = pos2 < CAP
    b1 = i1 * CAP
    b2 = i2 * CAP
    drow1 = jnp.where(keep1, b1 + pos1, TRASH)
    drow2 = jnp.where(keep2, b2 + pos2, TRASH)
    crow1 = b1 + jnp.minimum(pos1, CAP - 1)
    crow2 = b2 + jnp.minimum(pos2, CAP - 1)

    meta_ref[...] = (jnp.where(lane == 0, drow1, 0)
                     + jnp.where(lane == 1, drow2, 0)
                     + jnp.where(lane == 2, crow1, 0)
                     + jnp.where(lane == 3, crow2, 0))
    gm1 = jnp.where(keep1, g1, 0.0)
    gm2 = jnp.where(keep2, g2, 0.0)
    gb0_ref[...] = jnp.broadcast_to(gm1, (T, 16))
    gb1_ref[...] = jnp.broadcast_to(gm2, (T, 16))


_router_call = pl.pallas_call(
    _router_body,
    out_shape=[
        jax.ShapeDtypeStruct((T, E), jnp.int32),
        jax.ShapeDtypeStruct((T, 16), jnp.float32),
        jax.ShapeDtypeStruct((T, 16), jnp.float32),
    ],
)


_sc_mesh = plsc.VectorSubcoreMesh(core_axis_name="c", subcore_axis_name="s")


@functools.partial(
    pl.kernel,
    mesh=_sc_mesh,
    out_type=jax.ShapeDtypeStruct((TBL, H), jnp.float32),
    scratch_types=[
        pltpu.VMEM((TPW, H), jnp.float32),
        pltpu.VMEM((TPW,), jnp.int32),
        pltpu.VMEM((TPW,), jnp.int32),
        pltpu.SemaphoreType.DMA,
    ],
)
def _dispatch(x_hbm, d0_hbm, d1_hbm, tbl_hbm, xbuf, idx0, idx1, sem):
    wid = lax.axis_index("s") * 2 + lax.axis_index("c")
    base = wid * TPW
    pltpu.sync_copy(x_hbm.at[pl.ds(base, TPW)], xbuf)
    pltpu.sync_copy(d0_hbm.at[pl.ds(base, TPW)], idx0)
    pltpu.sync_copy(d1_hbm.at[pl.ds(base, TPW)], idx1)
    cp0 = pltpu.async_copy(xbuf, tbl_hbm.at[idx0], sem)
    cp1 = pltpu.async_copy(xbuf, tbl_hbm.at[idx1], sem)
    cp0.wait()
    cp1.wait()


def _gemm_body(x_ref, w1_ref, w2_ref, out_ref, acc_ref):
    fb = pl.program_id(1)
    m = pl.program_id(2)
    x = x_ref[...].astype(jnp.bfloat16)
    hid = jax.lax.dot(x, w1_ref[0].astype(jnp.bfloat16),
                      preferred_element_type=jnp.float32)
    hid = jax.nn.gelu(hid).astype(jnp.bfloat16)
    contrib = jax.lax.dot(hid, w2_ref[0].astype(jnp.bfloat16),
                          preferred_element_type=jnp.float32)

    @pl.when(fb == 0)
    def _():
        acc_ref[pl.ds(m * BM, BM), :] = contrib

    @pl.when(fb == FB - 1)
    def _():
        out_ref[...] = acc_ref[pl.ds(m * BM, BM), :] + contrib


_gemm_call = pl.pallas_call(
    _gemm_body,
    grid=(E, FB, M_BLKS),
    in_specs=[
        pl.BlockSpec((BM, H), lambda e, fb, m: (e * M_BLKS + m, 0)),
        pl.BlockSpec((1, H, F_BLK), lambda e, fb, m: (e, 0, fb)),
        pl.BlockSpec((1, F_BLK, H), lambda e, fb, m: (e, fb, 0)),
    ],
    out_specs=pl.BlockSpec((BM, H), lambda e, fb, m: (e * M_BLKS + m, 0)),
    out_shape=jax.ShapeDtypeStruct((E * CAP, H), jnp.float32),
    scratch_shapes=[pltpu.VMEM((CAP, H), jnp.float32)],
)


@functools.partial(
    pl.kernel,
    mesh=_sc_mesh,
    out_type=jax.ShapeDtypeStruct((T, H), jnp.float32),
    scratch_types=[
        pltpu.VMEM((SUB, H), jnp.float32),
        pltpu.VMEM((SUB, H), jnp.float32),
        pltpu.VMEM((SUB,), jnp.int32),
        pltpu.VMEM((SUB,), jnp.int32),
        pltpu.VMEM((SUB, 16), jnp.float32),
        pltpu.VMEM((SUB, 16), jnp.float32),
        pltpu.SemaphoreType.DMA,
    ],
)
def _combine(eout_hbm, c0_hbm, c1_hbm, gb0_hbm, gb1_hbm, out_hbm,
             b0, b1, i0, i1, g0, g1, sem):
    wid = lax.axis_index("s") * 2 + lax.axis_index("c")
    for sub in range(TPW // SUB):
        base = wid * TPW + sub * SUB
        pltpu.sync_copy(c0_hbm.at[pl.ds(base, SUB)], i0)
        pltpu.sync_copy(c1_hbm.at[pl.ds(base, SUB)], i1)
        pltpu.sync_copy(gb0_hbm.at[pl.ds(base, SUB)], g0)
        pltpu.sync_copy(gb1_hbm.at[pl.ds(base, SUB)], g1)
        cp0 = pltpu.async_copy(eout_hbm.at[i0], b0, sem)
        cp1 = pltpu.async_copy(eout_hbm.at[i1], b1, sem)
        cp0.wait()
        cp1.wait()

        def row(j, _):
            s0 = g0[j]
            s1 = g1[j]

            def col(c, _):
                sl = pl.ds(c * 16, 16)
                b0[j, sl] = s0 * b0[j, sl] + s1 * b1[j, sl]
                return 0

            lax.fori_loop(0, H // 16, col, 0, unroll=8)
            return 0

        lax.fori_loop(0, SUB, row, 0)
        pltpu.sync_copy(b0, out_hbm.at[pl.ds(base, SUB)])


def kernel(input, router_weight, w1, w2):
    s, b, h = input.shape
    xf = input.reshape(T, H)
    meta, gb0, gb1 = _router_call(xf, router_weight)
    tbl = _dispatch(xf, meta[:, 0], meta[:, 1])
    eout = _gemm_call(tbl, w1, w2)
    out = _combine(eout, meta[:, 2], meta[:, 3], gb0, gb1)
    return out.reshape(s, b, h)
